# baseline (speedup 1.0000x reference)
import functools

import jax
import jax.numpy as jnp
from jax import lax
from jax.experimental import pallas as pl
from jax.experimental.pallas import tpu as pltpu
from jax.experimental.pallas import tpu_sc as plsc

VOCAB = 50257
HIDDEN = 1024
BATCH = 4
SEQ = 8192

_info = plsc.get_sparse_core_info()
NC, NS = _info.num_cores, _info.num_subcores
NW = NC * NS
TOK_PER_W = (BATCH * SEQ) // NW
CHUNK = 32
NCHUNK = TOK_PER_W // CHUNK
W_PER_ROW = SEQ // TOK_PER_W


def _body(ids_hbm, tok_hbm, pos_hbm, out_hbm, idx_v, rows_v, pos_v, sem, psem):
    wid = lax.axis_index("s") * NC + lax.axis_index("c")
    base = wid * TOK_PER_W
    pos_base = lax.rem(wid, W_PER_ROW) * TOK_PER_W
    pltpu.sync_copy(ids_hbm.at[pl.ds(base, TOK_PER_W)], idx_v)

    def chunk_body(c, carry):
        off = c * CHUNK
        pcp = pltpu.async_copy(
            pos_hbm.at[pl.ds(pos_base + off, CHUNK)], pos_v, psem
        )
        pltpu.async_copy(
            tok_hbm.at[idx_v.at[pl.ds(off, CHUNK)]], rows_v, sem
        ).wait()
        pcp.wait()

        def add_row(r, inner):
            for j in range(HIDDEN // 16):
                sl = pl.ds(j * 16, 16)
                plsc.addupdate(rows_v.at[r, sl], pos_v[r, sl])
            return inner

        lax.fori_loop(0, CHUNK, add_row, 0)
        pltpu.sync_copy(rows_v, out_hbm.at[pl.ds(base + off, CHUNK)])
        return carry

    lax.fori_loop(0, NCHUNK, chunk_body, 0)


@jax.jit
def _embed(ids_flat, token_table, pos_table):
    mesh = plsc.VectorSubcoreMesh(core_axis_name="c", subcore_axis_name="s")
    k = pl.kernel(
        _body,
        out_type=jax.ShapeDtypeStruct((BATCH * SEQ, HIDDEN), jnp.float32),
        mesh=mesh,
        scratch_types=[
            pltpu.VMEM((TOK_PER_W,), jnp.int32),
            pltpu.VMEM((CHUNK, HIDDEN), jnp.float32),
            pltpu.VMEM((CHUNK, HIDDEN), jnp.float32),
            pltpu.SemaphoreType.DMA,
            pltpu.SemaphoreType.DMA,
        ],
    )
    return k(ids_flat, token_table, pos_table)


def kernel(input_ids, token_table, pos_table):
    ids_flat = input_ids.reshape(BATCH * SEQ).astype(jnp.int32)
    out = _embed(ids_flat, token_table, pos_table)
    return out.reshape(BATCH, SEQ, HIDDEN)


# speedup vs baseline: 1.3970x; 1.3970x over previous
import jax
import jax.numpy as jnp
from jax import lax
from jax.experimental import pallas as pl
from jax.experimental.pallas import tpu as pltpu
from jax.experimental.pallas import tpu_sc as plsc

VOCAB = 50257
HIDDEN = 1024
BATCH = 4
SEQ = 8192

_info = plsc.get_sparse_core_info()
NC, NS = _info.num_cores, _info.num_subcores
NW = NC * NS
POS_PER_W = SEQ // NW
PC = 16
NPC = POS_PER_W // PC
NG = NPC // 2
LANES = 16


def _body(ids_hbm, tok_hbm, pos_hbm, out_hbm,
          idx_v, pos_b, rows_b, gsems, osems, psems):
    wid = lax.axis_index("s") * NC + lax.axis_index("c")
    s0 = wid * POS_PER_W

    for b in range(BATCH):
        pltpu.sync_copy(ids_hbm.at[b, pl.ds(s0, POS_PER_W)], idx_v.at[b])

    def gather_cp(g, k):
        pcl, b = divmod(k, BATCH)
        pc = 2 * g + pcl
        return pltpu.make_async_copy(
            tok_hbm.at[idx_v.at[b, pl.ds(pc * PC, PC)]],
            rows_b.at[b], gsems.at[b])

    def out_cp(g, k):
        pcl, b = divmod(k, BATCH)
        pc = 2 * g + pcl
        return pltpu.make_async_copy(
            rows_b.at[b],
            out_hbm.at[pl.ds(b * SEQ + s0 + pc * PC, PC)],
            osems.at[b])

    def pos_cp(pc, pb):
        return pltpu.make_async_copy(
            pos_hbm.at[pl.ds(s0 + pc * PC, PC)], pos_b.at[pb], psems.at[pb])

    def add_rows(rbuf, pbuf):
        def add_row(r, carry):
            for j in range(HIDDEN // LANES):
                sl = pl.ds(j * LANES, LANES)
                plsc.addupdate(rbuf.at[r, sl], pbuf[r, sl])
            return carry

        lax.fori_loop(0, PC, add_row, 0)

    pos_cp(0, 0).start()
    pos_cp(1, 1).start()
    gather_cp(0, 0).start()

    def iter_body(g, carry):
        for k in range(2 * BATCH):
            pcl, b = divmod(k, BATCH)
            if k == 0:
                pos_cp(2 * g, 0).wait()
            if k == BATCH:
                pos_cp(2 * g + 1, 1).wait()

                @pl.when(g + 1 < NG)
                def _():
                    pos_cp(2 * g + 2, 0).start()

            if k < 2 * BATCH - 1:
                if k + 1 >= BATCH:
                    out_cp(g, k + 1 - BATCH).wait()
                else:
                    @pl.when(g > 0)
                    def _():
                        out_cp(g - 1, k + 1 + BATCH).wait()

                gather_cp(g, k + 1).start()
            gather_cp(g, k).wait()
            add_rows(rows_b.at[b], pos_b.at[pcl])
            out_cp(g, k).start()
            if k == 2 * BATCH - 1:
                @pl.when(g + 1 < NG)
                def _():
                    pos_cp(2 * g + 3, 1).start()
                    out_cp(g, BATCH).wait()
                    gather_cp(g + 1, 0).start()

        return carry

    lax.fori_loop(0, NG, iter_body, 0)
    for k in range(BATCH + 1, 2 * BATCH):
        out_cp(NG - 1, k).wait()


@jax.jit
def _embed(input_ids, token_table, pos_table):
    mesh = plsc.VectorSubcoreMesh(core_axis_name="c", subcore_axis_name="s")
    k = pl.kernel(
        _body,
        out_type=jax.ShapeDtypeStruct((BATCH * SEQ, HIDDEN), jnp.float32),
        mesh=mesh,
        scratch_types=[
            pltpu.VMEM((BATCH, POS_PER_W), jnp.int32),
            pltpu.VMEM((2, PC, HIDDEN), jnp.float32),
            pltpu.VMEM((BATCH, PC, HIDDEN), jnp.float32),
            pltpu.SemaphoreType.DMA((BATCH,)),
            pltpu.SemaphoreType.DMA((BATCH,)),
            pltpu.SemaphoreType.DMA((2,)),
        ],
    )
    return k(input_ids, token_table, pos_table)


def kernel(input_ids, token_table, pos_table):
    out = _embed(input_ids.astype(jnp.int32), token_table, pos_table)
    return out.reshape(BATCH, SEQ, HIDDEN)


# speedup vs baseline: 1.8519x; 1.3256x over previous
import jax
import jax.numpy as jnp
from jax import lax
from jax.experimental import pallas as pl
from jax.experimental.pallas import tpu as pltpu
from jax.experimental.pallas import tpu_sc as plsc

VOCAB = 50257
HIDDEN = 1024
BATCH = 4
SEQ = 8192

_info = plsc.get_sparse_core_info()
NC, NS = _info.num_cores, _info.num_subcores
NW = NC * NS
POS_PER_W = SEQ // NW
PC = 16
NPC = POS_PER_W // PC
NG = NPC // 2
LANES = 16


def _body(ids_hbm, tok_hbm, pos_hbm, out_hbm,
          idx_v, pos_b, rows_b, gsems, osems, psems):
    wid = lax.axis_index("s") * NC + lax.axis_index("c")
    s0 = wid * POS_PER_W

    for b in range(BATCH):
        pltpu.sync_copy(ids_hbm.at[b, pl.ds(s0, POS_PER_W)], idx_v.at[b])

    def gather_cp(g, k):
        pcl, b = divmod(k, BATCH)
        pc = 2 * g + pcl
        return pltpu.make_async_copy(
            tok_hbm.at[idx_v.at[b, pl.ds(pc * PC, PC)]],
            rows_b.at[b], gsems.at[b])

    def out_cp(g, k):
        pcl, b = divmod(k, BATCH)
        pc = 2 * g + pcl
        return pltpu.make_async_copy(
            rows_b.at[b],
            out_hbm.at[pl.ds(b * SEQ + s0 + pc * PC, PC)],
            osems.at[b])

    def pos_cp(pc, pb):
        return pltpu.make_async_copy(
            pos_hbm.at[pl.ds(s0 + pc * PC, PC)], pos_b.at[pb], psems.at[pb])

    def add_rows(rbuf, pbuf):
        @plsc.parallel_loop(0, PC, unroll=2)
        def _(r):
            for j in range(HIDDEN // LANES):
                sl = pl.ds(j * LANES, LANES)
                plsc.addupdate(rbuf.at[r, sl], pbuf[r, sl])

    pos_cp(0, 0).start()
    pos_cp(1, 1).start()
    gather_cp(0, 0).start()

    def iter_body(g, carry):
        for k in range(2 * BATCH):
            pcl, b = divmod(k, BATCH)
            if k == 0:
                pos_cp(2 * g, 0).wait()
            if k == BATCH:
                pos_cp(2 * g + 1, 1).wait()

                @pl.when(g + 1 < NG)
                def _():
                    pos_cp(2 * g + 2, 0).start()

            if k < 2 * BATCH - 1:
                if k + 1 >= BATCH:
                    out_cp(g, k + 1 - BATCH).wait()
                else:
                    @pl.when(g > 0)
                    def _():
                        out_cp(g - 1, k + 1 + BATCH).wait()

                gather_cp(g, k + 1).start()
            gather_cp(g, k).wait()
            add_rows(rows_b.at[b], pos_b.at[pcl])
            out_cp(g, k).start()
            if k == 2 * BATCH - 1:
                @pl.when(g + 1 < NG)
                def _():
                    pos_cp(2 * g + 3, 1).start()
                    out_cp(g, BATCH).wait()
                    gather_cp(g + 1, 0).start()

        return carry

    lax.fori_loop(0, NG, iter_body, 0)
    for k in range(BATCH + 1, 2 * BATCH):
        out_cp(NG - 1, k).wait()


@jax.jit
def _embed(input_ids, token_table, pos_table):
    mesh = plsc.VectorSubcoreMesh(core_axis_name="c", subcore_axis_name="s")
    k = pl.kernel(
        _body,
        out_type=jax.ShapeDtypeStruct((BATCH * SEQ, HIDDEN), jnp.float32),
        mesh=mesh,
        scratch_types=[
            pltpu.VMEM((BATCH, POS_PER_W), jnp.int32),
            pltpu.VMEM((2, PC, HIDDEN), jnp.float32),
            pltpu.VMEM((BATCH, PC, HIDDEN), jnp.float32),
            pltpu.SemaphoreType.DMA((BATCH,)),
            pltpu.SemaphoreType.DMA((BATCH,)),
            pltpu.SemaphoreType.DMA((2,)),
        ],
    )
    return k(input_ids, token_table, pos_table)


def kernel(input_ids, token_table, pos_table):
    out = _embed(input_ids.astype(jnp.int32), token_table, pos_table)
    return out.reshape(BATCH, SEQ, HIDDEN)


# speedup vs baseline: 2.0258x; 1.0939x over previous
import jax
import jax.numpy as jnp
from jax import lax
from jax.experimental import pallas as pl
from jax.experimental.pallas import tpu as pltpu
from jax.experimental.pallas import tpu_sc as plsc

VOCAB = 50257
HIDDEN = 1024
BATCH = 4
SEQ = 8192

_info = plsc.get_sparse_core_info()
NC, NS = _info.num_cores, _info.num_subcores
NW = NC * NS
POS_PER_W = SEQ // NW
PC = 16
NPC = POS_PER_W // PC
NG = NPC // 2
LANES = 16


def _body(ids_hbm, tok_hbm, pos_hbm, out_hbm,
          idx_v, pos_b, rows_b, gsems, osems, psems):
    wid = lax.axis_index("s") * NC + lax.axis_index("c")
    s0 = wid * POS_PER_W

    for b in range(BATCH):
        pltpu.sync_copy(ids_hbm.at[b, pl.ds(s0, POS_PER_W)], idx_v.at[b])

    def gather_cp(g, k):
        pcl, b = divmod(k, BATCH)
        pc = 2 * g + pcl
        return pltpu.make_async_copy(
            tok_hbm.at[idx_v.at[b, pl.ds(pc * PC, PC)]],
            rows_b.at[b], gsems.at[b])

    def out_cp(g, k):
        pcl, b = divmod(k, BATCH)
        pc = 2 * g + pcl
        return pltpu.make_async_copy(
            rows_b.at[b],
            out_hbm.at[pl.ds(b * SEQ + s0 + pc * PC, PC)],
            osems.at[b])

    def pos_cp(pc, pb):
        return pltpu.make_async_copy(
            pos_hbm.at[pl.ds(s0 + pc * PC, PC)], pos_b.at[pb], psems.at[pb])

    def add_rows(rbuf, pbuf):
        @plsc.parallel_loop(0, PC, unroll=2)
        def _(r):
            for j in range(HIDDEN // LANES):
                sl = pl.ds(j * LANES, LANES)
                plsc.addupdate(rbuf.at[r, sl], pbuf[r, sl])

    pos_cp(0, 0).start()
    pos_cp(1, 1).start()
    gather_cp(0, 0).start()
    gather_cp(0, 1).start()

    def iter_body(g, carry):
        for k in range(2 * BATCH):
            pcl, b = divmod(k, BATCH)
            if k == 0:
                pos_cp(2 * g, 0).wait()
            if k == BATCH:
                pos_cp(2 * g + 1, 1).wait()

                @pl.when(g + 1 < NG)
                def _():
                    pos_cp(2 * g + 2, 0).start()

            if k >= 2:
                out_cp(g, k - 2).wait()
            else:
                @pl.when(g > 0)
                def _():
                    out_cp(g - 1, k + 2 * BATCH - 2).wait()

            if k < 2 * BATCH - 2:
                gather_cp(g, k + 2).start()
            else:
                @pl.when(g + 1 < NG)
                def _():
                    gather_cp(g + 1, k - (2 * BATCH - 2)).start()

            gather_cp(g, k).wait()
            add_rows(rows_b.at[b], pos_b.at[pcl])
            out_cp(g, k).start()
            if k == 2 * BATCH - 1:
                @pl.when(g + 1 < NG)
                def _():
                    pos_cp(2 * g + 3, 1).start()

        return carry

    lax.fori_loop(0, NG, iter_body, 0)
    for k in range(2 * BATCH - 2, 2 * BATCH):
        out_cp(NG - 1, k).wait()


@jax.jit
def _embed(input_ids, token_table, pos_table):
    mesh = plsc.VectorSubcoreMesh(core_axis_name="c", subcore_axis_name="s")
    k = pl.kernel(
        _body,
        out_type=jax.ShapeDtypeStruct((BATCH * SEQ, HIDDEN), jnp.float32),
        mesh=mesh,
        scratch_types=[
            pltpu.VMEM((BATCH, POS_PER_W), jnp.int32),
            pltpu.VMEM((2, PC, HIDDEN), jnp.float32),
            pltpu.VMEM((BATCH, PC, HIDDEN), jnp.float32),
            pltpu.SemaphoreType.DMA((BATCH,)),
            pltpu.SemaphoreType.DMA((BATCH,)),
            pltpu.SemaphoreType.DMA((2,)),
        ],
    )
    return k(input_ids, token_table, pos_table)


def kernel(input_ids, token_table, pos_table):
    out = _embed(input_ids.astype(jnp.int32), token_table, pos_table)
    return out.reshape(BATCH, SEQ, HIDDEN)


# speedup vs baseline: 2.4358x; 1.2024x over previous
import jax
import jax.numpy as jnp
from jax import lax
from jax.experimental import pallas as pl
from jax.experimental.pallas import tpu as pltpu
from jax.experimental.pallas import tpu_sc as plsc

VOCAB = 50257
HIDDEN = 1024
BATCH = 4
SEQ = 8192

_info = plsc.get_sparse_core_info()
NC, NS = _info.num_cores, _info.num_subcores
NW = NC * NS
POS_PER_W = SEQ // NW
PC = 16
NPC = POS_PER_W // PC
NG = NPC // 2
LANES = 16
IDX_PER_W = POS_PER_W * BATCH


def _body(idsr_hbm, tok_hbm, pos_hbm, out_hbm,
          idx_v, pos_b, rows_b, gsems, osems, psems):
    wid = lax.axis_index("s") * NC + lax.axis_index("c")
    s0 = wid * POS_PER_W

    pltpu.sync_copy(idsr_hbm.at[pl.ds(wid * IDX_PER_W, IDX_PER_W)], idx_v)

    def gather_cp(g, u):
        pcl, bp = divmod(u, 2)
        off = (2 * g + pcl) * (2 * PC * 2) + bp * (2 * PC)
        return pltpu.make_async_copy(
            tok_hbm.at[idx_v.at[pl.ds(off, 2 * PC)]],
            rows_b.at[bp], gsems.at[bp])

    def out_cps(g, u):
        pcl, bp = divmod(u, 2)
        pc = 2 * g + pcl
        return [
            pltpu.make_async_copy(
                rows_b.at[bp, pl.ds(h * PC, PC)],
                out_hbm.at[pl.ds((2 * bp + h) * SEQ + s0 + pc * PC, PC)],
                osems.at[bp])
            for h in range(2)
        ]

    def pos_cp(pc, pb):
        return pltpu.make_async_copy(
            pos_hbm.at[pl.ds(s0 + pc * PC, PC)], pos_b.at[pb], psems.at[pb])

    def add_rows(rbuf, pbuf):
        @plsc.parallel_loop(0, PC, unroll=2)
        def _(r):
            for j in range(HIDDEN // LANES):
                sl = pl.ds(j * LANES, LANES)
                x = pbuf[r, sl]
                plsc.addupdate(rbuf.at[r, sl], x)
                plsc.addupdate(rbuf.at[PC + r, sl], x)

    pos_cp(0, 0).start()
    pos_cp(1, 1).start()
    gather_cp(0, 0).start()

    def iter_body(g, carry):
        for u in range(4):
            pcl, bp = divmod(u, 2)
            if u == 0:
                pos_cp(2 * g, 0).wait()
            if u == 2:
                pos_cp(2 * g + 1, 1).wait()

                @pl.when(g + 1 < NG)
                def _():
                    pos_cp(2 * g + 2, 0).start()

            if u > 0:
                for cp in out_cps(g, u - 1):
                    cp.wait()
            else:
                @pl.when(g > 0)
                def _():
                    for cp in out_cps(g - 1, 3):
                        cp.wait()

            if u < 3:
                gather_cp(g, u + 1).start()
            else:
                @pl.when(g + 1 < NG)
                def _():
                    gather_cp(g + 1, 0).start()

            gather_cp(g, u).wait()
            add_rows(rows_b.at[bp], pos_b.at[pcl])
            for cp in out_cps(g, u):
                cp.start()
            if u == 3:
                @pl.when(g + 1 < NG)
                def _():
                    pos_cp(2 * g + 3, 1).start()

        return carry

    lax.fori_loop(0, NG, iter_body, 0)
    for cp in out_cps(NG - 1, 3):
        cp.wait()


@jax.jit
def _embed(ids_re, token_table, pos_table):
    mesh = plsc.VectorSubcoreMesh(core_axis_name="c", subcore_axis_name="s")
    k = pl.kernel(
        _body,
        out_type=jax.ShapeDtypeStruct((BATCH * SEQ, HIDDEN), jnp.float32),
        mesh=mesh,
        scratch_types=[
            pltpu.VMEM((IDX_PER_W,), jnp.int32),
            pltpu.VMEM((2, PC, HIDDEN), jnp.float32),
            pltpu.VMEM((2, 2 * PC, HIDDEN), jnp.float32),
            pltpu.SemaphoreType.DMA((2,)),
            pltpu.SemaphoreType.DMA((2,)),
            pltpu.SemaphoreType.DMA((2,)),
        ],
    )
    return k(ids_re, token_table, pos_table)


def kernel(input_ids, token_table, pos_table):
    ids_re = (
        input_ids.astype(jnp.int32)
        .reshape(BATCH, SEQ // PC, PC)
        .transpose(1, 0, 2)
        .reshape(BATCH * SEQ)
    )
    out = _embed(ids_re, token_table, pos_table)
    return out.reshape(BATCH, SEQ, HIDDEN)
